# baseline (speedup 1.0000x reference)
import functools

import jax
import jax.numpy as jnp
from jax import lax
from jax.experimental import pallas as pl
from jax.experimental.pallas import tpu as pltpu
from jax.experimental.pallas import tpu_sc as plsc

N = 50000
E = 1600000
G = 512
F = 32
NP = 50048
RPT = NP // 16
NW = 32
EPW_ROWS = 396
E_PAD = NW * EPW_ROWS * 128
CH = 3
NCH = EPW_ROWS // CH
IDXR = E_PAD // 128
IDXR_PAD = IDXR + 8

RPW = 1664
NP2 = NW * RPW
SEG_ROWS = 520
SEG_FLAT = SEG_ROWS * F


def _sc_mesh():
  return plsc.VectorSubcoreMesh(core_axis_name="c", subcore_axis_name="s")


@functools.partial(
    pl.kernel,
    out_type=jax.ShapeDtypeStruct((2 * NP, F), jnp.float32),
    mesh=_sc_mesh(),
    scratch_types=[
        pltpu.VMEM_SHARED((NP, F), jnp.float32),
        pltpu.VMEM((CH, 128), jnp.int32),
        pltpu.VMEM((CH, 128), jnp.int32),
        pltpu.VMEM((CH, 128), jnp.int32),
        pltpu.VMEM((CH, 128), jnp.int32),
        pltpu.VMEM((CH * 128, F), jnp.float32),
        pltpu.VMEM((CH * 128, F), jnp.float32),
        pltpu.VMEM((64, F), jnp.float32),
        pltpu.SemaphoreType.DMA,
        pltpu.SemaphoreType.DMA,
        pltpu.SemaphoreType.DMA,
        pltpu.SemaphoreType.DMA,
        pltpu.SemaphoreType.DMA,
        pltpu.SemaphoreType.DMA,
        pltpu.SemaphoreType.DMA,
        pltpu.SemaphoreType.DMA,
        pltpu.SemaphoreType.DMA,
    ],
    compiler_params=pltpu.CompilerParams(use_tc_tiling_on_sc=False),
)
def _sc_scatter(z_hbm, srcm_hbm, dstm_hbm, zrow_hbm, out_hbm,
                accum, sbA, sbB, dbA, dbB, rowsA, rowsB, zbuf,
                gsA, gsB, ssA, ssB, isA, isB, idA, idB, zsem):
  c = lax.axis_index("c")
  s = lax.axis_index("s")
  wid = c * 16 + s
  base = s * RPT
  row0 = wid * EPW_ROWS

  A = (sbA, dbA, rowsA, gsA, ssA, isA, idA)
  B = (sbB, dbB, rowsB, gsB, ssB, isB, idB)

  pltpu.sync_copy(zrow_hbm, zbuf)
  zc = [pltpu.async_copy(zbuf, accum.at[pl.ds(base + k * 64, 64)], zsem)
        for k in range(RPT // 64)]
  zc.append(pltpu.async_copy(zbuf.at[pl.ds(0, RPT % 64)],
                             accum.at[pl.ds(base + (RPT // 64) * 64, RPT % 64)],
                             zsem))

  def fire_gathers(X, _cc):
    sb, _, rows, gs, _, _, _ = X
    for j in range(CH):
      pltpu.async_copy(z_hbm.at[sb.at[j]], rows.at[pl.ds(j * 128, 128)], gs)

  def wait_gathers(X):
    sb, _, rows, gs, _, _, _ = X
    for j in range(CH):
      pltpu.make_async_copy(z_hbm.at[sb.at[j]],
                            rows.at[pl.ds(j * 128, 128)], gs).wait()

  def fire_scatters(X):
    _, db, rows, _, ss, _, _ = X
    for j in range(CH):
      pltpu.async_copy(rows.at[pl.ds(j * 128, 128)], accum.at[db.at[j]], ss,
                       add=True)

  def wait_scatters(X):
    _, db, rows, _, ss, _, _ = X
    for j in range(CH):
      pltpu.make_async_copy(rows.at[pl.ds(j * 128, 128)],
                            accum.at[db.at[j]], ss).wait()

  def fire_src(X, cc):
    sb, _, _, _, _, isem, _ = X
    pltpu.async_copy(srcm_hbm.at[pl.ds(row0 + cc * CH, CH)], sb, isem)

  def wait_src(X, cc):
    sb, _, _, _, _, isem, _ = X
    pltpu.make_async_copy(srcm_hbm.at[pl.ds(row0 + cc * CH, CH)],
                          sb, isem).wait()

  def fire_dst(X, cc):
    _, db, _, _, _, _, idsem = X
    pltpu.async_copy(dstm_hbm.at[pl.ds(row0 + cc * CH, CH)], db, idsem)

  def wait_dst(X, cc):
    _, db, _, _, _, _, idsem = X
    pltpu.make_async_copy(dstm_hbm.at[pl.ds(row0 + cc * CH, CH)],
                          db, idsem).wait()

  pltpu.sync_copy(srcm_hbm.at[pl.ds(row0, CH)], sbA)
  pltpu.sync_copy(dstm_hbm.at[pl.ds(row0, CH)], dbA)
  fire_gathers(A, 0)
  for h in zc:
    h.wait()
  plsc.subcore_barrier()
  wait_gathers(A)
  fire_scatters(A)
  fire_src(A, 2)
  fire_dst(B, 1)
  fire_src(B, 1)
  wait_src(B, 1)
  fire_gathers(B, 1)

  def phase(cc, X, Y):
    wait_gathers(X)
    wait_dst(X, cc)
    fire_scatters(X)
    fire_src(X, cc + 2)
    wait_scatters(Y)
    fire_dst(Y, cc + 1)
    wait_src(Y, cc + 1)
    fire_gathers(Y, cc + 1)

  def pair_body(k, carry):
    phase(2 * k + 1, B, A)
    phase(2 * k + 2, A, B)
    return carry

  lax.fori_loop(0, (NCH - 2) // 2, pair_body, 0)

  last = NCH - 1
  wait_gathers(B)
  wait_dst(B, last)
  fire_scatters(B)
  wait_scatters(A)
  wait_scatters(B)
  wait_src(A, NCH)
  plsc.subcore_barrier()

  pltpu.sync_copy(accum.at[pl.ds(base, RPT)],
                  out_hbm.at[pl.ds(c * NP + base, RPT)])


@functools.partial(
    pl.kernel,
    out_type=jax.ShapeDtypeStruct((NW * SEG_FLAT,), jnp.float32),
    mesh=_sc_mesh(),
    scratch_types=[
        pltpu.VMEM((128 * F,), jnp.float32),
        pltpu.VMEM((128 * 16,), jnp.int32),
        pltpu.VMEM((SEG_FLAT,), jnp.float32),
    ],
    compiler_params=pltpu.CompilerParams(use_tc_tiling_on_sc=False,
                                         needs_layout_passes=False),
)
def _sc_segmax(h3f_hbm, bbf_hbm, out_hbm, hbuf, bbuf, outloc):
  c = lax.axis_index("c")
  s = lax.axis_index("s")
  wid = c * 16 + s
  row0 = wid * RPW

  minf = jnp.full((16,), -jnp.inf, jnp.float32)
  lane = lax.iota(jnp.int32, 16)

  def init_body(i, carry):
    outloc[pl.ds(i * 16, 16)] = minf
    return carry

  lax.fori_loop(0, SEG_FLAT // 16, init_body, 0)

  def chunk_body(ci, carry):
    r = row0 + ci * 128
    pltpu.sync_copy(h3f_hbm.at[pl.ds(r * F, 128 * F)], hbuf)
    pltpu.sync_copy(bbf_hbm.at[pl.ds(r * 16, 128 * 16)], bbuf)

    def row_body(i, rc):
      prev, alo, ahi = rc
      bv = bbuf[pl.ds(i * 16, 16)]
      rlo = hbuf[pl.ds(i * F, 16)]
      rhi = hbuf[pl.ds(i * F + 16, 16)]
      newseg = bv != prev
      alo = jnp.maximum(jnp.where(newseg, minf, alo), rlo)
      ahi = jnp.maximum(jnp.where(newseg, minf, ahi), rhi)
      idx = bv * F + lane
      plsc.store_scatter(outloc, [idx], alo)
      plsc.store_scatter(outloc, [idx + 16], ahi)
      return (bv, alo, ahi)

    return lax.fori_loop(0, 128, row_body, carry)

  prev0 = jnp.full((16,), -1, jnp.int32)
  lax.fori_loop(0, RPW // 128, chunk_body, (prev0, minf, minf))

  pltpu.sync_copy(outloc, out_hbm.at[pl.ds(wid * SEG_FLAT, SEG_FLAT)])


PR = NP // 4
_BR = 3128
_NB = PR // _BR

_row = lambda i: (i, 0)
_p0 = lambda i: (i, 0)
_p1 = lambda i: (i + _NB, 0)
_w = lambda i: (0, 0)


def _row_specs(n_rowlike, n_big, n_small=0):
  specs = [pl.BlockSpec((_BR, 128), _p0), pl.BlockSpec((_BR, 128), _p1)]
  specs += [pl.BlockSpec((_BR, 128), _row) for _ in range(n_rowlike)]
  specs += [pl.BlockSpec((128, 128), _w) for _ in range(n_big)]
  specs += [pl.BlockSpec((1, 128), _w) for _ in range(n_small)]
  return specs


def _gin1_body(p0, p1, x, w, sel, b, h_s, dinvb):
  ps = p0[...] + p1[...]
  degb = jnp.dot(ps, sel[...], preferred_element_type=jnp.float32) + 1.0
  dvb = lax.rsqrt(degb)
  h = jnp.maximum(
      jnp.dot(x[...] + ps, w[...],
              preferred_element_type=jnp.float32) + b[...], 0.0)
  h_s[...] = dvb * h
  dinvb[...] = dvb


def _gin1(pflat, x_aug, w1p, selp, b1p):
  return pl.pallas_call(
      _gin1_body,
      grid=(_NB,),
      in_specs=_row_specs(1, 2, 1),
      out_specs=(pl.BlockSpec((_BR, 128), _row),
                 pl.BlockSpec((_BR, 128), _row)),
      out_shape=(jax.ShapeDtypeStruct((PR, 128), jnp.float32),
                 jax.ShapeDtypeStruct((PR, 128), jnp.float32)),
  )(pflat, pflat, x_aug, w1p, selp, b1p)


def _combine_mid_body(p0, p1, z, dinvb, out):
  d2 = dinvb[...] * dinvb[...]
  out[...] = d2 * (p0[...] + p1[...] + z[...])


def _combine_mid(pflat, z, dinvb):
  return pl.pallas_call(
      _combine_mid_body,
      grid=(_NB,),
      in_specs=_row_specs(2, 0),
      out_specs=pl.BlockSpec((_BR, 128), _row),
      out_shape=jax.ShapeDtypeStruct((PR, 128), jnp.float32),
  )(pflat, pflat, z, dinvb)


def _combine_lin_body(post_dinv, p0, p1, z, dinvb, w, b, out):
  t = dinvb[...] * (p0[...] + p1[...] + z[...])
  h = jnp.dot(t, w[...], preferred_element_type=jnp.float32) + b[...]
  out[...] = dinvb[...] * h if post_dinv else h


def _combine_lin(pflat, z, dinvb, w, b, post_dinv):
  return pl.pallas_call(
      functools.partial(_combine_lin_body, post_dinv),
      grid=(_NB,),
      in_specs=_row_specs(2, 1, 1),
      out_specs=pl.BlockSpec((_BR, 128), _row),
      out_shape=jax.ShapeDtypeStruct((PR, 128), jnp.float32),
  )(pflat, pflat, z, dinvb, w, b)


def _gin2_body(p0, p1, h2, w, b, out):
  out[...] = jnp.maximum(
      jnp.dot(h2[...] + p0[...] + p1[...], w[...],
              preferred_element_type=jnp.float32) + b[...], 0.0)


def _gin2(pflat, h2, w2p, b2p):
  return pl.pallas_call(
      _gin2_body,
      grid=(_NB,),
      in_specs=_row_specs(1, 1, 1),
      out_specs=pl.BlockSpec((_BR, 128), _row),
      out_shape=jax.ShapeDtypeStruct((PR, 128), jnp.float32),
  )(pflat, pflat, h2, w2p, b2p)


def _final_body(parts, wf, bf, out):
  pooled = jnp.max(parts[...], axis=0)[:G]
  logits = jnp.dot(pooled, wf[...],
                   preferred_element_type=jnp.float32) + bf[...]
  col = lax.broadcasted_iota(jnp.int32, (G, 128), 1)
  valid = col < 3
  masked = jnp.where(valid, logits, -jnp.inf)
  m = jnp.max(masked, axis=1, keepdims=True)
  e = jnp.where(valid, jnp.exp(logits - m), 0.0)
  lse = jnp.log(jnp.sum(e, axis=1, keepdims=True))
  out[...] = logits - m - lse


def _final(parts3, wfp, bfp):
  return pl.pallas_call(
      _final_body,
      out_shape=jax.ShapeDtypeStruct((G, 128), jnp.float32),
  )(parts3, wfp, bfp)


def kernel(x, edge_index, batch, W1, b1, Ws1, bs1, Ws2, bs2, W2, b2, Wf, bf):
  f32 = jnp.float32

  xx = jnp.concatenate(
      [x.astype(f32), jnp.ones((N, 1), f32), jnp.zeros((N, F - 6), f32)],
      axis=1)
  x_aug = jnp.concatenate(
      [xx.reshape(N // 4, 128), jnp.zeros((PR - N // 4, 128), f32)])

  eflat = edge_index.astype(jnp.int32).reshape(-1)
  src = eflat[:E]
  dst = eflat[E:]
  pad = E_PAD - E
  fill = jax.lax.iota(jnp.int32, pad) % 16
  extra = jnp.zeros((IDXR_PAD - IDXR) * 128, jnp.int32)
  srcm = jnp.concatenate([src, fill, extra]).reshape(IDXR_PAD, 128)
  dstm = jnp.concatenate([dst, N + fill, extra]).reshape(IDXR_PAD, 128)
  zrow = jnp.zeros((64, F), f32)

  eye4 = jnp.eye(4, dtype=f32)

  def padw(w, b):
    wp = jnp.zeros((F, F), f32).at[:w.shape[0], :w.shape[1]].set(w)
    bp = jnp.zeros((1, F), f32).at[0, :b.shape[0]].set(b)
    return jnp.kron(eye4, wp), jnp.tile(bp, (1, 4))

  w1p, b1p = padw(W1, b1)
  ws1p, bs1p = padw(Ws1, bs1)
  ws2p, bs2p = padw(Ws2, bs2)
  w2p, b2p = padw(W2, b2)
  selp = jnp.kron(eye4, jnp.zeros((F, F), f32).at[5, :].set(1.0))
  wfp = jnp.zeros((F, 128), f32).at[:Wf.shape[0], :3].set(Wf)
  bfp = jnp.zeros((1, 128), f32).at[0, :3].set(bf)

  bid = jnp.full((NP2,), G, jnp.int32).at[:N].set(batch.astype(jnp.int32))
  bbf = jnp.broadcast_to(bid.reshape(NP2 // 8, 8, 1),
                         (NP2 // 8, 8, 16)).reshape(NP2 // 8, 128).reshape(-1)

  def scatter_p(zp):
    parts = _sc_scatter(zp.reshape(NP, F), srcm, dstm, zrow)
    return parts.reshape(2 * PR, 128)

  parts = scatter_p(x_aug)
  z, dinvb = _gin1(parts, x_aug, w1p, selp, b1p)

  for _ in range(4):
    z = _combine_mid(scatter_p(z), z, dinvb)
  z = _combine_lin(scatter_p(z), z, dinvb, ws1p, bs1p, post_dinv=True)

  for _ in range(4):
    z = _combine_mid(scatter_p(z), z, dinvb)
  h2 = _combine_lin(scatter_p(z), z, dinvb, ws2p, bs2p, post_dinv=False)

  h3 = _gin2(scatter_p(h2), h2, w2p, b2p)

  h3f = jnp.pad(h3, ((0, (NP2 - NP) // 4), (0, 0))).reshape(-1)
  segparts = _sc_segmax(h3f, bbf)

  out = _final(segparts.reshape(NW, SEG_ROWS, F), wfp, bfp)
  return out[:, :3]


# speedup vs baseline: 1.0400x; 1.0400x over previous
import functools

import jax
import jax.numpy as jnp
from jax import lax
from jax.experimental import pallas as pl
from jax.experimental.pallas import tpu as pltpu
from jax.experimental.pallas import tpu_sc as plsc

N = 50000
E = 1600000
G = 512
F = 32
NP = 50048
RPT = NP // 16
NW = 32
EPW_ROWS = 396
E_PAD = NW * EPW_ROWS * 128
CH = 3
NCH = EPW_ROWS // CH
IDXR = E_PAD // 128
IDXR_PAD = IDXR + 8

RPW = 1664
NP2 = NW * RPW
SEG_ROWS = 520
SEG_FLAT = SEG_ROWS * F


def _sc_mesh():
  return plsc.VectorSubcoreMesh(core_axis_name="c", subcore_axis_name="s")


@functools.partial(
    pl.kernel,
    out_type=jax.ShapeDtypeStruct((2 * NP, F), jnp.float32),
    mesh=_sc_mesh(),
    scratch_types=[
        pltpu.VMEM_SHARED((NP, F), jnp.float32),
        pltpu.VMEM((CH, 128), jnp.int32),
        pltpu.VMEM((CH, 128), jnp.int32),
        pltpu.VMEM((CH, 128), jnp.int32),
        pltpu.VMEM((CH, 128), jnp.int32),
        pltpu.VMEM((CH * 128, F), jnp.float32),
        pltpu.VMEM((CH * 128, F), jnp.float32),
        pltpu.VMEM((64, F), jnp.float32),
        pltpu.SemaphoreType.DMA,
        pltpu.SemaphoreType.DMA,
        pltpu.SemaphoreType.DMA,
        pltpu.SemaphoreType.DMA,
        pltpu.SemaphoreType.DMA,
        pltpu.SemaphoreType.DMA,
        pltpu.SemaphoreType.DMA,
        pltpu.SemaphoreType.DMA,
        pltpu.SemaphoreType.DMA,
    ],
    compiler_params=pltpu.CompilerParams(use_tc_tiling_on_sc=False),
)
def _sc_scatter(z_hbm, srcm_hbm, dstm_hbm, zrow_hbm, out_hbm,
                accum, sbA, sbB, dbA, dbB, rowsA, rowsB, zbuf,
                gsA, gsB, ssA, ssB, isA, isB, idA, idB, zsem):
  c = lax.axis_index("c")
  s = lax.axis_index("s")
  wid = c * 16 + s
  base = s * RPT
  row0 = wid * EPW_ROWS

  A = (sbA, dbA, rowsA, gsA, ssA, isA, idA)
  B = (sbB, dbB, rowsB, gsB, ssB, isB, idB)

  pltpu.sync_copy(zrow_hbm, zbuf)
  zc = [pltpu.async_copy(zbuf, accum.at[pl.ds(base + k * 64, 64)], zsem)
        for k in range(RPT // 64)]
  zc.append(pltpu.async_copy(zbuf.at[pl.ds(0, RPT % 64)],
                             accum.at[pl.ds(base + (RPT // 64) * 64, RPT % 64)],
                             zsem))

  def fire_gathers(X, _cc):
    sb, _, rows, gs, _, _, _ = X
    for j in range(CH):
      pltpu.async_copy(z_hbm.at[sb.at[j]], rows.at[pl.ds(j * 128, 128)], gs)

  def wait_gathers(X):
    sb, _, rows, gs, _, _, _ = X
    for j in range(CH):
      pltpu.make_async_copy(z_hbm.at[sb.at[j]],
                            rows.at[pl.ds(j * 128, 128)], gs).wait()

  def fire_scatters(X):
    _, db, rows, _, ss, _, _ = X
    for j in range(CH):
      pltpu.async_copy(rows.at[pl.ds(j * 128, 128)], accum.at[db.at[j]], ss,
                       add=True)

  def wait_scatters(X):
    _, db, rows, _, ss, _, _ = X
    for j in range(CH):
      pltpu.make_async_copy(rows.at[pl.ds(j * 128, 128)],
                            accum.at[db.at[j]], ss).wait()

  def fire_src(X, cc):
    sb, _, _, _, _, isem, _ = X
    pltpu.async_copy(srcm_hbm.at[pl.ds(row0 + cc * CH, CH)], sb, isem)

  def wait_src(X, cc):
    sb, _, _, _, _, isem, _ = X
    pltpu.make_async_copy(srcm_hbm.at[pl.ds(row0 + cc * CH, CH)],
                          sb, isem).wait()

  def fire_dst(X, cc):
    _, db, _, _, _, _, idsem = X
    pltpu.async_copy(dstm_hbm.at[pl.ds(row0 + cc * CH, CH)], db, idsem)

  def wait_dst(X, cc):
    _, db, _, _, _, _, idsem = X
    pltpu.make_async_copy(dstm_hbm.at[pl.ds(row0 + cc * CH, CH)],
                          db, idsem).wait()

  pltpu.sync_copy(srcm_hbm.at[pl.ds(row0, CH)], sbA)
  pltpu.sync_copy(dstm_hbm.at[pl.ds(row0, CH)], dbA)
  fire_gathers(A, 0)
  for h in zc:
    h.wait()
  plsc.subcore_barrier()
  wait_gathers(A)
  fire_scatters(A)
  fire_src(A, 2)
  fire_dst(B, 1)
  fire_src(B, 1)
  wait_src(B, 1)
  fire_gathers(B, 1)

  def phase(cc, X, Y):
    wait_gathers(X)
    wait_dst(X, cc)
    fire_scatters(X)
    fire_src(X, cc + 2)
    wait_scatters(Y)
    fire_dst(Y, cc + 1)
    wait_src(Y, cc + 1)
    fire_gathers(Y, cc + 1)

  def pair_body(k, carry):
    phase(2 * k + 1, B, A)
    phase(2 * k + 2, A, B)
    return carry

  lax.fori_loop(0, (NCH - 2) // 2, pair_body, 0)

  last = NCH - 1
  wait_gathers(B)
  wait_dst(B, last)
  fire_scatters(B)
  wait_scatters(A)
  wait_scatters(B)
  wait_src(A, NCH)
  plsc.subcore_barrier()

  pltpu.sync_copy(accum.at[pl.ds(base, RPT)],
                  out_hbm.at[pl.ds(c * NP + base, RPT)])


@functools.partial(
    pl.kernel,
    out_type=jax.ShapeDtypeStruct((NW * SEG_FLAT,), jnp.float32),
    mesh=_sc_mesh(),
    scratch_types=[
        pltpu.VMEM((128 * F,), jnp.float32),
        pltpu.VMEM((128 * 16,), jnp.int32),
        pltpu.VMEM((SEG_FLAT,), jnp.float32),
    ],
    compiler_params=pltpu.CompilerParams(use_tc_tiling_on_sc=False,
                                         needs_layout_passes=False),
)
def _sc_segmax(h3f_hbm, bbf_hbm, out_hbm, hbuf, bbuf, outloc):
  c = lax.axis_index("c")
  s = lax.axis_index("s")
  wid = c * 16 + s
  row0 = wid * RPW

  minf = jnp.full((16,), -jnp.inf, jnp.float32)
  lane = lax.iota(jnp.int32, 16)

  def init_body(i, carry):
    outloc[pl.ds(i * 16, 16)] = minf
    return carry

  lax.fori_loop(0, SEG_FLAT // 16, init_body, 0)

  def chunk_body(ci, carry):
    r = row0 + ci * 128
    pltpu.sync_copy(h3f_hbm.at[pl.ds(r * F, 128 * F)], hbuf)
    pltpu.sync_copy(bbf_hbm.at[pl.ds(r * 16, 128 * 16)], bbuf)

    def row_body(i, rc):
      prev, alo, ahi = rc
      bv = bbuf[pl.ds(i * 16, 16)]
      rlo = hbuf[pl.ds(i * F, 16)]
      rhi = hbuf[pl.ds(i * F + 16, 16)]
      newseg = bv != prev
      alo = jnp.maximum(jnp.where(newseg, minf, alo), rlo)
      ahi = jnp.maximum(jnp.where(newseg, minf, ahi), rhi)
      idx = bv * F + lane
      plsc.store_scatter(outloc, [idx], alo)
      plsc.store_scatter(outloc, [idx + 16], ahi)
      return (bv, alo, ahi)

    return lax.fori_loop(0, 128, row_body, carry)

  prev0 = jnp.full((16,), -1, jnp.int32)
  lax.fori_loop(0, RPW // 128, chunk_body, (prev0, minf, minf))

  pltpu.sync_copy(outloc, out_hbm.at[pl.ds(wid * SEG_FLAT, SEG_FLAT)])


PR = NP // 4
_BR = 3128
_NB = PR // _BR

_row = lambda i: (i, 0)
_p0 = lambda i: (i, 0)
_p1 = lambda i: (i + _NB, 0)
_w = lambda i: (0, 0)


def _row_specs(n_rowlike, n_big, n_small=0):
  specs = [pl.BlockSpec((_BR, 128), _p0), pl.BlockSpec((_BR, 128), _p1)]
  specs += [pl.BlockSpec((_BR, 128), _row) for _ in range(n_rowlike)]
  specs += [pl.BlockSpec((128, 128), _w) for _ in range(n_big)]
  specs += [pl.BlockSpec((1, 128), _w) for _ in range(n_small)]
  return specs


def _gin1_body(p0, p1, x, w, sel, b, h_s, dinvb):
  ps = p0[...] + p1[...]
  degb = jnp.dot(ps, sel[...], preferred_element_type=jnp.float32) + 1.0
  dvb = lax.rsqrt(degb)
  h = jnp.maximum(
      jnp.dot(x[...] + ps, w[...],
              preferred_element_type=jnp.float32) + b[...], 0.0)
  h_s[...] = dvb * h
  dinvb[...] = dvb


def _gin1(pflat, x_aug, w1p, selp, b1p):
  return pl.pallas_call(
      _gin1_body,
      grid=(_NB,),
      in_specs=_row_specs(1, 2, 1),
      out_specs=(pl.BlockSpec((_BR, 128), _row),
                 pl.BlockSpec((_BR, 128), _row)),
      out_shape=(jax.ShapeDtypeStruct((PR, 128), jnp.float32),
                 jax.ShapeDtypeStruct((PR, 128), jnp.float32)),
  )(pflat, pflat, x_aug, w1p, selp, b1p)


def _combine_mid_body(p0, p1, z, dinvb, out):
  d2 = dinvb[...] * dinvb[...]
  out[...] = d2 * (p0[...] + p1[...] + z[...])


def _combine_mid(pflat, z, dinvb):
  return pl.pallas_call(
      _combine_mid_body,
      grid=(_NB,),
      in_specs=_row_specs(2, 0),
      out_specs=pl.BlockSpec((_BR, 128), _row),
      out_shape=jax.ShapeDtypeStruct((PR, 128), jnp.float32),
  )(pflat, pflat, z, dinvb)


def _combine_lin_body(post_dinv, p0, p1, z, dinvb, w, b, out):
  t = dinvb[...] * (p0[...] + p1[...] + z[...])
  h = jnp.dot(t, w[...], preferred_element_type=jnp.float32) + b[...]
  out[...] = dinvb[...] * h if post_dinv else h


def _combine_lin(pflat, z, dinvb, w, b, post_dinv):
  return pl.pallas_call(
      functools.partial(_combine_lin_body, post_dinv),
      grid=(_NB,),
      in_specs=_row_specs(2, 1, 1),
      out_specs=pl.BlockSpec((_BR, 128), _row),
      out_shape=jax.ShapeDtypeStruct((PR, 128), jnp.float32),
  )(pflat, pflat, z, dinvb, w, b)


def _gin2_body(p0, p1, h2, w, b, out):
  out[...] = jnp.maximum(
      jnp.dot(h2[...] + p0[...] + p1[...], w[...],
              preferred_element_type=jnp.float32) + b[...], 0.0)


def _gin2(pflat, h2, w2p, b2p):
  return pl.pallas_call(
      _gin2_body,
      grid=(_NB,),
      in_specs=_row_specs(1, 1, 1),
      out_specs=pl.BlockSpec((_BR, 128), _row),
      out_shape=jax.ShapeDtypeStruct((PR, 128), jnp.float32),
  )(pflat, pflat, h2, w2p, b2p)


def _final_body(parts, wf, bf, out):
  pooled = jnp.max(parts[...], axis=0)[:G]
  logits = jnp.dot(pooled, wf[...],
                   preferred_element_type=jnp.float32) + bf[...]
  col = lax.broadcasted_iota(jnp.int32, (G, 128), 1)
  valid = col < 3
  masked = jnp.where(valid, logits, -jnp.inf)
  m = jnp.max(masked, axis=1, keepdims=True)
  e = jnp.where(valid, jnp.exp(logits - m), 0.0)
  lse = jnp.log(jnp.sum(e, axis=1, keepdims=True))
  out[...] = logits - m - lse


def _final(parts3, wfp, bfp):
  return pl.pallas_call(
      _final_body,
      out_shape=jax.ShapeDtypeStruct((G, 128), jnp.float32),
  )(parts3, wfp, bfp)


def kernel(x, edge_index, batch, W1, b1, Ws1, bs1, Ws2, bs2, W2, b2, Wf, bf):
  f32 = jnp.float32

  xx = jnp.concatenate(
      [x.astype(f32), jnp.ones((N, 1), f32), jnp.zeros((N, F - 6), f32)],
      axis=1)
  x_aug = jnp.concatenate(
      [xx.reshape(N // 4, 128), jnp.zeros((PR - N // 4, 128), f32)])

  src = edge_index[0].astype(jnp.int32)
  dst = edge_index[1].astype(jnp.int32)
  pad = E_PAD - E
  fill = jax.lax.iota(jnp.int32, pad) % 16
  extra = jnp.zeros((IDXR_PAD - IDXR) * 128, jnp.int32)
  srcm = jnp.concatenate([src, fill, extra]).reshape(IDXR_PAD, 128)
  dstm = jnp.concatenate([dst, N + fill, extra]).reshape(IDXR_PAD, 128)
  zrow = jnp.zeros((64, F), f32)

  eye4 = jnp.eye(4, dtype=f32)

  def padw(w, b):
    wp = jnp.zeros((F, F), f32).at[:w.shape[0], :w.shape[1]].set(w)
    bp = jnp.zeros((1, F), f32).at[0, :b.shape[0]].set(b)
    return jnp.kron(eye4, wp), jnp.tile(bp, (1, 4))

  w1p, b1p = padw(W1, b1)
  ws1p, bs1p = padw(Ws1, bs1)
  ws2p, bs2p = padw(Ws2, bs2)
  w2p, b2p = padw(W2, b2)
  selp = jnp.kron(eye4, jnp.zeros((F, F), f32).at[5, :].set(1.0))
  wfp = jnp.zeros((F, 128), f32).at[:Wf.shape[0], :3].set(Wf)
  bfp = jnp.zeros((1, 128), f32).at[0, :3].set(bf)

  bid = jnp.full((NP2,), G, jnp.int32).at[:N].set(batch.astype(jnp.int32))
  bbf = jnp.broadcast_to(bid[:, None], (NP2, 16)).reshape(-1)

  def scatter_p(zp):
    parts = _sc_scatter(zp.reshape(NP, F), srcm, dstm, zrow)
    return parts.reshape(2 * PR, 128)

  parts = scatter_p(x_aug)
  z, dinvb = _gin1(parts, x_aug, w1p, selp, b1p)

  for _ in range(4):
    z = _combine_mid(scatter_p(z), z, dinvb)
  z = _combine_lin(scatter_p(z), z, dinvb, ws1p, bs1p, post_dinv=True)

  for _ in range(4):
    z = _combine_mid(scatter_p(z), z, dinvb)
  h2 = _combine_lin(scatter_p(z), z, dinvb, ws2p, bs2p, post_dinv=False)

  h3 = _gin2(scatter_p(h2), h2, w2p, b2p)

  h3f = jnp.concatenate([h3.reshape(-1), jnp.zeros((NP2 - NP) * F, f32)])
  segparts = _sc_segmax(h3f, bbf)

  out = _final(segparts.reshape(NW, SEG_ROWS, F), wfp, bfp)
  return out[:, :3]


# speedup vs baseline: 1.0439x; 1.0037x over previous
import functools

import jax
import jax.numpy as jnp
from jax import lax
from jax.experimental import pallas as pl
from jax.experimental.pallas import tpu as pltpu
from jax.experimental.pallas import tpu_sc as plsc

N = 50000
E = 1600000
G = 512
F = 32
NP = 50048
RPT = NP // 16
NW = 32
EPW_ROWS = 396
E_PAD = NW * EPW_ROWS * 128
CH = 3
NCH = EPW_ROWS // CH
IDXR = E_PAD // 128
IDXR_PAD = IDXR + 8

RPW = 1664
NP2 = NW * RPW
SEG_ROWS = 520
SEG_FLAT = SEG_ROWS * F


def _sc_mesh():
  return plsc.VectorSubcoreMesh(core_axis_name="c", subcore_axis_name="s")


@functools.partial(
    pl.kernel,
    out_type=jax.ShapeDtypeStruct((2 * NP, F), jnp.float32),
    mesh=_sc_mesh(),
    scratch_types=[
        pltpu.VMEM_SHARED((NP, F), jnp.float32),
        pltpu.VMEM((CH, 128), jnp.int32),
        pltpu.VMEM((CH, 128), jnp.int32),
        pltpu.VMEM((CH, 128), jnp.int32),
        pltpu.VMEM((CH, 128), jnp.int32),
        pltpu.VMEM((CH * 128, F), jnp.float32),
        pltpu.VMEM((CH * 128, F), jnp.float32),
        pltpu.VMEM((64, F), jnp.float32),
        pltpu.SemaphoreType.DMA,
        pltpu.SemaphoreType.DMA,
        pltpu.SemaphoreType.DMA,
        pltpu.SemaphoreType.DMA,
        pltpu.SemaphoreType.DMA,
        pltpu.SemaphoreType.DMA,
        pltpu.SemaphoreType.DMA,
        pltpu.SemaphoreType.DMA,
        pltpu.SemaphoreType.DMA,
    ],
    compiler_params=pltpu.CompilerParams(use_tc_tiling_on_sc=False),
)
def _sc_scatter(z_hbm, srcm_hbm, dstm_hbm, zrow_hbm, out_hbm,
                accum, sbA, sbB, dbA, dbB, rowsA, rowsB, zbuf,
                gsA, gsB, ssA, ssB, isA, isB, idA, idB, zsem):
  c = lax.axis_index("c")
  s = lax.axis_index("s")
  wid = c * 16 + s
  base = s * RPT
  row0 = wid * EPW_ROWS

  A = (sbA, dbA, rowsA, gsA, ssA, isA, idA)
  B = (sbB, dbB, rowsB, gsB, ssB, isB, idB)

  pltpu.sync_copy(zrow_hbm, zbuf)
  zc = [pltpu.async_copy(zbuf, accum.at[pl.ds(base + k * 64, 64)], zsem)
        for k in range(RPT // 64)]
  zc.append(pltpu.async_copy(zbuf.at[pl.ds(0, RPT % 64)],
                             accum.at[pl.ds(base + (RPT // 64) * 64, RPT % 64)],
                             zsem))

  def fire_gathers(X, _cc):
    sb, _, rows, gs, _, _, _ = X
    for j in range(CH):
      pltpu.async_copy(z_hbm.at[sb.at[j]], rows.at[pl.ds(j * 128, 128)], gs)

  def wait_gathers(X):
    sb, _, rows, gs, _, _, _ = X
    for j in range(CH):
      pltpu.make_async_copy(z_hbm.at[sb.at[j]],
                            rows.at[pl.ds(j * 128, 128)], gs).wait()

  def fire_scatters(X):
    _, db, rows, _, ss, _, _ = X
    for j in range(CH):
      pltpu.async_copy(rows.at[pl.ds(j * 128, 128)], accum.at[db.at[j]], ss,
                       add=True)

  def wait_scatters(X):
    _, db, rows, _, ss, _, _ = X
    for j in range(CH):
      pltpu.make_async_copy(rows.at[pl.ds(j * 128, 128)],
                            accum.at[db.at[j]], ss).wait()

  def fire_src(X, cc):
    sb, _, _, _, _, isem, _ = X
    pltpu.async_copy(srcm_hbm.at[pl.ds(row0 + cc * CH, CH)], sb, isem)

  def wait_src(X, cc):
    sb, _, _, _, _, isem, _ = X
    pltpu.make_async_copy(srcm_hbm.at[pl.ds(row0 + cc * CH, CH)],
                          sb, isem).wait()

  def fire_dst(X, cc):
    _, db, _, _, _, _, idsem = X
    pltpu.async_copy(dstm_hbm.at[pl.ds(row0 + cc * CH, CH)], db, idsem)

  def wait_dst(X, cc):
    _, db, _, _, _, _, idsem = X
    pltpu.make_async_copy(dstm_hbm.at[pl.ds(row0 + cc * CH, CH)],
                          db, idsem).wait()

  pltpu.sync_copy(srcm_hbm.at[pl.ds(row0, CH)], sbA)
  pltpu.sync_copy(dstm_hbm.at[pl.ds(row0, CH)], dbA)
  fire_gathers(A, 0)
  for h in zc:
    h.wait()
  plsc.subcore_barrier()
  wait_gathers(A)
  fire_scatters(A)
  fire_src(A, 2)
  fire_dst(B, 1)
  fire_src(B, 1)
  wait_src(B, 1)
  fire_gathers(B, 1)

  def phase(cc, X, Y):
    wait_gathers(X)
    wait_dst(X, cc)
    fire_scatters(X)
    fire_src(X, cc + 2)
    wait_scatters(Y)
    fire_dst(Y, cc + 1)
    wait_src(Y, cc + 1)
    fire_gathers(Y, cc + 1)

  def pair_body(k, carry):
    phase(2 * k + 1, B, A)
    phase(2 * k + 2, A, B)
    return carry

  lax.fori_loop(0, (NCH - 2) // 2, pair_body, 0)

  last = NCH - 1
  wait_gathers(B)
  wait_dst(B, last)
  fire_scatters(B)
  wait_scatters(A)
  wait_scatters(B)
  wait_src(A, NCH)
  plsc.subcore_barrier()

  pltpu.sync_copy(accum.at[pl.ds(base, RPT)],
                  out_hbm.at[pl.ds(c * NP + base, RPT)])


@functools.partial(
    pl.kernel,
    out_type=jax.ShapeDtypeStruct((NW * SEG_FLAT,), jnp.float32),
    mesh=_sc_mesh(),
    scratch_types=[
        pltpu.VMEM((128 * F,), jnp.float32),
        pltpu.VMEM((128,), jnp.int32),
        pltpu.VMEM((SEG_FLAT,), jnp.float32),
    ],
    compiler_params=pltpu.CompilerParams(use_tc_tiling_on_sc=False,
                                         needs_layout_passes=False),
)
def _sc_segmax(h3f_hbm, bid_hbm, out_hbm, hbuf, bbuf, outloc):
  c = lax.axis_index("c")
  s = lax.axis_index("s")
  wid = c * 16 + s
  row0 = wid * RPW

  minf = jnp.full((16,), -jnp.inf, jnp.float32)
  lane = lax.iota(jnp.int32, 16)

  def init_body(i, carry):
    outloc[pl.ds(i * 16, 16)] = minf
    return carry

  lax.fori_loop(0, SEG_FLAT // 16, init_body, 0)

  def chunk_body(ci, carry):
    r = row0 + ci * 128
    pltpu.sync_copy(h3f_hbm.at[pl.ds(r * F, 128 * F)], hbuf)
    pltpu.sync_copy(bid_hbm.at[pl.ds(r, 128)], bbuf)

    def row_body(i, rc):
      prev, alo, ahi = rc
      bv = plsc.load_gather(bbuf, [jnp.full((16,), 0, jnp.int32) + i])
      rlo = hbuf[pl.ds(i * F, 16)]
      rhi = hbuf[pl.ds(i * F + 16, 16)]
      newseg = bv != prev
      alo = jnp.maximum(jnp.where(newseg, minf, alo), rlo)
      ahi = jnp.maximum(jnp.where(newseg, minf, ahi), rhi)
      idx = bv * F + lane
      plsc.store_scatter(outloc, [idx], alo)
      plsc.store_scatter(outloc, [idx + 16], ahi)
      return (bv, alo, ahi)

    return lax.fori_loop(0, 128, row_body, carry)

  prev0 = jnp.full((16,), -1, jnp.int32)

  @pl.when(wid < 30)
  def _():
    lax.fori_loop(0, RPW // 128, chunk_body, (prev0, minf, minf))

  @pl.when(wid == 30)
  def _():
    lax.fori_loop(0, (NP - 30 * RPW) // 128, chunk_body, (prev0, minf, minf))

  pltpu.sync_copy(outloc, out_hbm.at[pl.ds(wid * SEG_FLAT, SEG_FLAT)])


PR = NP // 4
_BR = 3128
_NB = PR // _BR

_row = lambda i: (i, 0)
_p0 = lambda i: (i, 0)
_p1 = lambda i: (i + _NB, 0)
_w = lambda i: (0, 0)


def _row_specs(n_rowlike, n_big, n_small=0):
  specs = [pl.BlockSpec((_BR, 128), _p0), pl.BlockSpec((_BR, 128), _p1)]
  specs += [pl.BlockSpec((_BR, 128), _row) for _ in range(n_rowlike)]
  specs += [pl.BlockSpec((128, 128), _w) for _ in range(n_big)]
  specs += [pl.BlockSpec((1, 128), _w) for _ in range(n_small)]
  return specs


def _gin1_body(p0, p1, x, w, sel, b, h_s, dinvb):
  ps = p0[...] + p1[...]
  degb = jnp.dot(ps, sel[...], preferred_element_type=jnp.float32) + 1.0
  dvb = lax.rsqrt(degb)
  h = jnp.maximum(
      jnp.dot(x[...] + ps, w[...],
              preferred_element_type=jnp.float32) + b[...], 0.0)
  h_s[...] = dvb * h
  dinvb[...] = dvb


def _gin1(pflat, x_aug, w1p, selp, b1p):
  return pl.pallas_call(
      _gin1_body,
      grid=(_NB,),
      in_specs=_row_specs(1, 2, 1),
      out_specs=(pl.BlockSpec((_BR, 128), _row),
                 pl.BlockSpec((_BR, 128), _row)),
      out_shape=(jax.ShapeDtypeStruct((PR, 128), jnp.float32),
                 jax.ShapeDtypeStruct((PR, 128), jnp.float32)),
  )(pflat, pflat, x_aug, w1p, selp, b1p)


def _combine_mid_body(p0, p1, z, dinvb, out):
  d2 = dinvb[...] * dinvb[...]
  out[...] = d2 * (p0[...] + p1[...] + z[...])


def _combine_mid(pflat, z, dinvb):
  return pl.pallas_call(
      _combine_mid_body,
      grid=(_NB,),
      in_specs=_row_specs(2, 0),
      out_specs=pl.BlockSpec((_BR, 128), _row),
      out_shape=jax.ShapeDtypeStruct((PR, 128), jnp.float32),
  )(pflat, pflat, z, dinvb)


def _combine_lin_body(post_dinv, p0, p1, z, dinvb, w, b, out):
  t = dinvb[...] * (p0[...] + p1[...] + z[...])
  h = jnp.dot(t, w[...], preferred_element_type=jnp.float32) + b[...]
  out[...] = dinvb[...] * h if post_dinv else h


def _combine_lin(pflat, z, dinvb, w, b, post_dinv):
  return pl.pallas_call(
      functools.partial(_combine_lin_body, post_dinv),
      grid=(_NB,),
      in_specs=_row_specs(2, 1, 1),
      out_specs=pl.BlockSpec((_BR, 128), _row),
      out_shape=jax.ShapeDtypeStruct((PR, 128), jnp.float32),
  )(pflat, pflat, z, dinvb, w, b)


def _gin2_body(p0, p1, h2, w, b, out):
  out[...] = jnp.maximum(
      jnp.dot(h2[...] + p0[...] + p1[...], w[...],
              preferred_element_type=jnp.float32) + b[...], 0.0)


def _gin2(pflat, h2, w2p, b2p):
  return pl.pallas_call(
      _gin2_body,
      grid=(_NB,),
      in_specs=_row_specs(1, 1, 1),
      out_specs=pl.BlockSpec((_BR, 128), _row),
      out_shape=jax.ShapeDtypeStruct((PR, 128), jnp.float32),
  )(pflat, pflat, h2, w2p, b2p)


def _final_body(parts, wf, bf, out):
  pooled = jnp.max(parts[...], axis=0)[:G]
  logits = jnp.dot(pooled, wf[...],
                   preferred_element_type=jnp.float32) + bf[...]
  col = lax.broadcasted_iota(jnp.int32, (G, 128), 1)
  valid = col < 3
  masked = jnp.where(valid, logits, -jnp.inf)
  m = jnp.max(masked, axis=1, keepdims=True)
  e = jnp.where(valid, jnp.exp(logits - m), 0.0)
  lse = jnp.log(jnp.sum(e, axis=1, keepdims=True))
  out[...] = logits - m - lse


def _final(parts3, wfp, bfp):
  return pl.pallas_call(
      _final_body,
      out_shape=jax.ShapeDtypeStruct((G, 128), jnp.float32),
  )(parts3, wfp, bfp)


def kernel(x, edge_index, batch, W1, b1, Ws1, bs1, Ws2, bs2, W2, b2, Wf, bf):
  f32 = jnp.float32

  xx = jnp.concatenate(
      [x.astype(f32), jnp.ones((N, 1), f32), jnp.zeros((N, F - 6), f32)],
      axis=1)
  x_aug = jnp.concatenate(
      [xx.reshape(N // 4, 128), jnp.zeros((PR - N // 4, 128), f32)])

  src = edge_index[0].astype(jnp.int32)
  dst = edge_index[1].astype(jnp.int32)
  pad = E_PAD - E
  fill = jax.lax.iota(jnp.int32, pad) % 16
  extra = jnp.zeros((IDXR_PAD - IDXR) * 128, jnp.int32)
  srcm = jnp.concatenate([src, fill, extra]).reshape(IDXR_PAD, 128)
  dstm = jnp.concatenate([dst, N + fill, extra]).reshape(IDXR_PAD, 128)
  zrow = jnp.zeros((64, F), f32)

  eye4 = jnp.eye(4, dtype=f32)

  def padw(w, b):
    wp = jnp.zeros((F, F), f32).at[:w.shape[0], :w.shape[1]].set(w)
    bp = jnp.zeros((1, F), f32).at[0, :b.shape[0]].set(b)
    return jnp.kron(eye4, wp), jnp.tile(bp, (1, 4))

  w1p, b1p = padw(W1, b1)
  ws1p, bs1p = padw(Ws1, bs1)
  ws2p, bs2p = padw(Ws2, bs2)
  w2p, b2p = padw(W2, b2)
  selp = jnp.kron(eye4, jnp.zeros((F, F), f32).at[5, :].set(1.0))
  wfp = jnp.zeros((F, 128), f32).at[:Wf.shape[0], :3].set(Wf)
  bfp = jnp.zeros((1, 128), f32).at[0, :3].set(bf)

  bid = jnp.full((NP,), G, jnp.int32).at[:N].set(batch.astype(jnp.int32))

  def scatter_p(zp):
    parts = _sc_scatter(zp.reshape(NP, F), srcm, dstm, zrow)
    return parts.reshape(2 * PR, 128)

  parts = scatter_p(x_aug)
  z, dinvb = _gin1(parts, x_aug, w1p, selp, b1p)

  for _ in range(4):
    z = _combine_mid(scatter_p(z), z, dinvb)
  z = _combine_lin(scatter_p(z), z, dinvb, ws1p, bs1p, post_dinv=True)

  for _ in range(4):
    z = _combine_mid(scatter_p(z), z, dinvb)
  h2 = _combine_lin(scatter_p(z), z, dinvb, ws2p, bs2p, post_dinv=False)

  h3 = _gin2(scatter_p(h2), h2, w2p, b2p)

  segparts = _sc_segmax(h3.reshape(-1), bid)

  out = _final(segparts.reshape(NW, SEG_ROWS, F), wfp, bfp)
  return out[:, :3]
